# baseline (speedup 1.0000x reference)
import functools

import jax
import jax.numpy as jnp
from jax import lax
from jax.experimental import pallas as pl
from jax.experimental.pallas import tpu as pltpu
from jax.experimental.pallas import tpu_sc as plsc

NC = 2
NS = 16
NW = NC * NS

IDX_W = 128
SUB = 8
CHUNK = IDX_W * SUB


@functools.partial(jax.jit, static_argnames=("n_per_w", "n_chunks"))
def _embed_call(idx2d, table, *, n_per_w, n_chunks):
    n, d = idx2d.shape[0] * idx2d.shape[1], table.shape[1]
    mesh = plsc.VectorSubcoreMesh(core_axis_name="c", subcore_axis_name="s")

    @functools.partial(
        pl.kernel,
        out_type=jax.ShapeDtypeStruct((n, d), table.dtype),
        mesh=mesh,
        scratch_types=[
            pltpu.VMEM((SUB, IDX_W), jnp.int32),
            pltpu.VMEM((CHUNK, d), jnp.int32),
            pltpu.SemaphoreType.DMA,
        ],
        compiler_params=pltpu.CompilerParams(use_tc_tiling_on_sc=False),
    )
    def body(idx_hbm, table_hbm, out_hbm, idx_v, rows_v, sem):
        wid = lax.axis_index("s") * NC + lax.axis_index("c")
        row_base = wid * n_per_w

        def chunk_body(g, carry):
            row0 = row_base + g * CHUNK
            pltpu.sync_copy(
                idx_hbm.at[pl.ds(pl.multiple_of(row0 // IDX_W, SUB), SUB)], idx_v
            )
            copies = [
                pltpu.async_copy(
                    table_hbm.at[idx_v.at[j]],
                    rows_v.at[pl.ds(j * IDX_W, IDX_W)],
                    sem,
                )
                for j in range(SUB)
            ]
            for c in copies:
                c.wait()
            pltpu.sync_copy(rows_v, out_hbm.at[pl.ds(row0, CHUNK)])
            return carry

        lax.fori_loop(0, n_chunks, chunk_body, 0)

    return body(idx2d, table)


def kernel(input_ids, embedding):
    b, s = input_ids.shape
    v, d = embedding.shape
    n = b * s
    assert n % (NW * CHUNK) == 0
    n_per_w = n // NW
    n_chunks = n_per_w // CHUNK
    idx2d = input_ids.reshape(n // IDX_W, IDX_W).astype(jnp.int32)
    table_i32 = jax.lax.bitcast_convert_type(
        embedding.reshape(v, d // 2, 2), jnp.int32
    )
    out = _embed_call(idx2d, table_i32, n_per_w=n_per_w, n_chunks=n_chunks)
    out_bf16 = jax.lax.bitcast_convert_type(out, embedding.dtype)
    return out_bf16.reshape(b, s, d)


# speedup vs baseline: 1.3118x; 1.3118x over previous
import functools

import jax
import jax.numpy as jnp
from jax import lax
from jax.experimental import pallas as pl
from jax.experimental.pallas import tpu as pltpu
from jax.experimental.pallas import tpu_sc as plsc

NC = 2
NS = 16
NW = NC * NS

RW = 16


@functools.partial(jax.jit, static_argnames=("rows_per_w",))
def _embed_call(ids, table_i32, *, rows_per_w):
    bsz, seq = ids.shape
    d2 = table_i32.shape[1]
    n_chunks = rows_per_w // RW
    mesh = plsc.VectorSubcoreMesh(core_axis_name="c", subcore_axis_name="s")

    @functools.partial(
        pl.kernel,
        out_type=jax.ShapeDtypeStruct((bsz, seq, d2), jnp.int32),
        mesh=mesh,
        scratch_types=[
            pltpu.VMEM((RW, seq), jnp.int32),
            pltpu.VMEM((RW, seq, d2), jnp.int32),
            pltpu.SemaphoreType.DMA,
        ],
        compiler_params=pltpu.CompilerParams(use_tc_tiling_on_sc=False),
    )
    def body(ids_hbm, table_hbm, out_hbm, idx_v, rows_v, sem):
        wid = lax.axis_index("s") * NC + lax.axis_index("c")
        row_base = wid * rows_per_w

        def chunk_body(g, carry):
            r0 = row_base + g * RW
            pltpu.sync_copy(ids_hbm.at[pl.ds(r0, RW)], idx_v)
            copies = [
                pltpu.async_copy(
                    table_hbm.at[idx_v.at[r]],
                    rows_v.at[r],
                    sem,
                )
                for r in range(RW)
            ]
            for c in copies:
                c.wait()
            pltpu.sync_copy(rows_v, out_hbm.at[pl.ds(r0, RW)])
            return carry

        lax.fori_loop(0, n_chunks, chunk_body, 0)

    return body(ids, table_i32)


def kernel(input_ids, embedding):
    b, s = input_ids.shape
    v, d = embedding.shape
    assert b % (NW * RW) == 0
    table_i32 = jax.lax.bitcast_convert_type(
        embedding.reshape(v, d // 2, 2), jnp.int32
    )
    out = _embed_call(
        input_ids.astype(jnp.int32), table_i32, rows_per_w=b // NW
    )
    return jax.lax.bitcast_convert_type(out, embedding.dtype).reshape(b, s, d)


# speedup vs baseline: 1.5577x; 1.1875x over previous
import functools

import jax
import jax.numpy as jnp
from jax import lax
from jax.experimental import pallas as pl
from jax.experimental.pallas import tpu as pltpu
from jax.experimental.pallas import tpu_sc as plsc

NC = 2
NS = 16
NW = NC * NS

RW = 16


@functools.partial(jax.jit, static_argnames=("rows_per_w",))
def _embed_call(ids, table_i32, *, rows_per_w):
    bsz, seq = ids.shape
    d2 = table_i32.shape[1]
    d = d2 * 2
    n_chunks = rows_per_w // RW
    mesh = plsc.VectorSubcoreMesh(core_axis_name="c", subcore_axis_name="s")

    @functools.partial(
        pl.kernel,
        out_type=jax.ShapeDtypeStruct((bsz, seq, d), jnp.bfloat16),
        mesh=mesh,
        scratch_types=[
            pltpu.VMEM((RW, seq), jnp.int32),
            pltpu.VMEM((RW, seq, d2), jnp.int32),
            pltpu.VMEM((RW, seq, d), jnp.bfloat16),
            pltpu.SemaphoreType.DMA,
        ],
        compiler_params=pltpu.CompilerParams(
            use_tc_tiling_on_sc=False, needs_layout_passes=False
        ),
    )
    def body(ids_hbm, table_hbm, out_hbm, idx_v, rows_v, rows_bf, sem):
        wid = lax.axis_index("s") * NC + lax.axis_index("c")
        row_base = wid * rows_per_w

        def chunk_body(g, carry):
            r0 = row_base + g * RW
            pltpu.sync_copy(ids_hbm.at[pl.ds(r0, RW)], idx_v)
            copies = [
                pltpu.async_copy(
                    table_hbm.at[idx_v.at[r]],
                    rows_v.at[r],
                    sem,
                )
                for r in range(RW)
            ]
            for c in copies:
                c.wait()

            def retype(q, carry2):
                r = q // seq
                t = q % seq
                for k in range(d2 // 16):
                    w = rows_v[r, t, pl.ds(k * 16, 16)]
                    rows_bf[r, t, pl.ds(k * 32, 32)] = plsc.bitcast(
                        w, jnp.bfloat16
                    )
                return carry2

            lax.fori_loop(0, RW * seq, retype, 0)
            pltpu.sync_copy(rows_bf, out_hbm.at[pl.ds(r0, RW)])
            return carry

        lax.fori_loop(0, n_chunks, chunk_body, 0)

    return body(ids, table_i32)


def kernel(input_ids, embedding):
    b, s = input_ids.shape
    v, d = embedding.shape
    assert b % (NW * RW) == 0
    table_i32 = jax.lax.bitcast_convert_type(
        embedding.reshape(v, d // 2, 2), jnp.int32
    )
    return _embed_call(
        input_ids.astype(jnp.int32), table_i32, rows_per_w=b // NW
    )


# speedup vs baseline: 2.2261x; 1.4291x over previous
import functools

import jax
import jax.numpy as jnp
from jax import lax
from jax.experimental import pallas as pl
from jax.experimental.pallas import tpu as pltpu
from jax.experimental.pallas import tpu_sc as plsc

NC = 2
NS = 16
NW = NC * NS

RW = 16
TK = 1250

_SC_PARAMS = pltpu.CompilerParams(
    use_tc_tiling_on_sc=False, needs_layout_passes=False
)


def _retype_table(table_bf16):
    v, d = table_bf16.shape
    d2 = d // 2
    rows_per_w = v // NW
    n_chunks = rows_per_w // TK
    mesh = plsc.VectorSubcoreMesh(core_axis_name="c", subcore_axis_name="s")

    @functools.partial(
        pl.kernel,
        out_type=jax.ShapeDtypeStruct((v, d2), jnp.int32),
        mesh=mesh,
        scratch_types=[
            pltpu.VMEM((TK, d), jnp.bfloat16),
            pltpu.VMEM((TK, d2), jnp.int32),
        ],
        compiler_params=_SC_PARAMS,
    )
    def body(tab_hbm, out_hbm, bf_v, i32_v):
        wid = lax.axis_index("s") * NC + lax.axis_index("c")
        row_base = wid * rows_per_w

        def chunk_body(g, carry):
            r0 = row_base + g * TK
            pltpu.sync_copy(tab_hbm.at[pl.ds(r0, TK)], bf_v)

            def retype(r, carry2):
                for k in range(d2 // 16):
                    w = bf_v[r, pl.ds(k * 32, 32)]
                    i32_v[r, pl.ds(k * 16, 16)] = plsc.bitcast(w, jnp.int32)
                return carry2

            lax.fori_loop(0, TK, retype, 0)
            pltpu.sync_copy(i32_v, out_hbm.at[pl.ds(r0, TK)])
            return carry

        lax.fori_loop(0, n_chunks, chunk_body, 0)

    return body(table_bf16)


def _embed_gather(ids, table_i32, *, rows_per_w):
    bsz, seq = ids.shape
    d2 = table_i32.shape[1]
    d = d2 * 2
    n_chunks = rows_per_w // RW
    mesh = plsc.VectorSubcoreMesh(core_axis_name="c", subcore_axis_name="s")

    @functools.partial(
        pl.kernel,
        out_type=jax.ShapeDtypeStruct((bsz, seq, d), jnp.bfloat16),
        mesh=mesh,
        scratch_types=[
            pltpu.VMEM((RW, seq), jnp.int32),
            pltpu.VMEM((RW, seq, d2), jnp.int32),
            pltpu.VMEM((RW, seq, d), jnp.bfloat16),
            pltpu.SemaphoreType.DMA,
        ],
        compiler_params=_SC_PARAMS,
    )
    def body(ids_hbm, table_hbm, out_hbm, idx_v, rows_v, rows_bf, sem):
        wid = lax.axis_index("s") * NC + lax.axis_index("c")
        row_base = wid * rows_per_w

        def chunk_body(g, carry):
            r0 = row_base + g * RW
            pltpu.sync_copy(ids_hbm.at[pl.ds(r0, RW)], idx_v)
            copies = [
                pltpu.async_copy(
                    table_hbm.at[idx_v.at[r]],
                    rows_v.at[r],
                    sem,
                )
                for r in range(RW)
            ]
            for c in copies:
                c.wait()

            def retype(q, carry2):
                r = q // seq
                t = q % seq
                for k in range(d2 // 16):
                    w = rows_v[r, t, pl.ds(k * 16, 16)]
                    rows_bf[r, t, pl.ds(k * 32, 32)] = plsc.bitcast(
                        w, jnp.bfloat16
                    )
                return carry2

            lax.fori_loop(0, RW * seq, retype, 0)
            pltpu.sync_copy(rows_bf, out_hbm.at[pl.ds(r0, RW)])
            return carry

        lax.fori_loop(0, n_chunks, chunk_body, 0)

    return body(ids, table_i32)


@functools.partial(jax.jit, static_argnames=("rows_per_w",))
def _embed_call(ids, table_bf16, *, rows_per_w):
    table_i32 = _retype_table(table_bf16)
    return _embed_gather(ids, table_i32, rows_per_w=rows_per_w)


def kernel(input_ids, embedding):
    b, s = input_ids.shape
    v, d = embedding.shape
    assert b % (NW * RW) == 0 and v % (NW * TK) == 0
    return _embed_call(
        input_ids.astype(jnp.int32), embedding, rows_per_w=b // NW
    )
